# Initial kernel scaffold; baseline (speedup 1.0000x reference)
import jax
import jax.numpy as jnp
from jax.experimental import pallas as pl


def kernel(x, W, P):
    raise NotImplementedError("write your pallas kernel here")


# speedup vs baseline: 3.5944x; 3.5944x over previous
import functools

import jax
import jax.numpy as jnp
from jax import lax
from jax.experimental import pallas as pl
from jax.experimental.pallas import tpu as pltpu
from jax.experimental.pallas import tpu_sc as plsc

BATCH = 4096
SEQ = 200
D = 64
NFLAT = BATCH * SEQ
NC, NS, LANES = 2, 16, 16
NW = NC * NS
PER_W = NFLAT // NW
IDX_MINOR = 100
IDX_ROWS_PER_W = PER_W // IDX_MINOR
CHUNK_ROWS = 400
CHUNK_IDX = CHUNK_ROWS // IDX_MINOR
NCHUNK = PER_W // CHUNK_ROWS
DV = D // LANES


def _sc_body(x_hbm, w_hbm, p_hbm, out_hbm, idx_v, rows_v, p_v, sem):
    wid = lax.axis_index("s") * NC + lax.axis_index("c")
    pltpu.sync_copy(p_hbm, p_v)
    pltpu.sync_copy(
        x_hbm.at[pl.ds(wid * IDX_ROWS_PER_W, IDX_ROWS_PER_W)], idx_v
    )
    base = wid * PER_W

    @pl.loop(0, NCHUNK)
    def _chunk(c):
        descs = []
        for j in range(CHUNK_IDX):
            descs.append(
                pltpu.async_copy(
                    w_hbm.at[idx_v.at[c * CHUNK_IDX + j]],
                    rows_v.at[pl.ds(j * IDX_MINOR, IDX_MINOR)],
                    sem,
                )
            )
        for d in descs:
            d.wait()

        @pl.loop(0, SEQ)
        def _add(r):
            for d in range(DV):
                sl = pl.ds(d * LANES, LANES)
                pv = p_v[r, sl]
                rows_v[r, sl] += pv
                rows_v[r + SEQ, sl] += pv

        pltpu.sync_copy(rows_v, out_hbm.at[pl.ds(base + c * CHUNK_ROWS, CHUNK_ROWS)])


_sc_kernel = functools.partial(
    pl.kernel,
    out_type=jax.ShapeDtypeStruct((NFLAT, D), jnp.float32),
    mesh=plsc.VectorSubcoreMesh(core_axis_name="c", subcore_axis_name="s"),
    scratch_types=[
        pltpu.VMEM((IDX_ROWS_PER_W, IDX_MINOR), jnp.int32),
        pltpu.VMEM((CHUNK_ROWS, D), jnp.float32),
        pltpu.VMEM((SEQ, D), jnp.float32),
        pltpu.SemaphoreType.DMA,
    ],
    compiler_params=pltpu.CompilerParams(use_tc_tiling_on_sc=False),
)(_sc_body)


@jax.jit
def kernel(x, W, P):
    xf = x.reshape(NFLAT // IDX_MINOR, IDX_MINOR)
    out = _sc_kernel(xf, W, P)
    return out.reshape(BATCH, SEQ, D)


# speedup vs baseline: 4.1247x; 1.1475x over previous
import functools

import jax
import jax.numpy as jnp
from jax import lax
from jax.experimental import pallas as pl
from jax.experimental.pallas import tpu as pltpu
from jax.experimental.pallas import tpu_sc as plsc

BATCH = 4096
SEQ = 200
D = 64
NFLAT = BATCH * SEQ
NC, NS, LANES = 2, 16, 16
NW = NC * NS
PER_W = NFLAT // NW
IDX_MINOR = 100
IDX_ROWS_PER_W = PER_W // IDX_MINOR
CHUNK_ROWS = SEQ
CHUNK_IDX = CHUNK_ROWS // IDX_MINOR
NCHUNK = PER_W // CHUNK_ROWS
DV = D // LANES
NBUF = 4


def _sc_body(x_hbm, w_hbm, p_hbm, out_hbm,
             idx_v, b0, b1, b2, b3, p_v,
             g0, g1, g2, g3, s0, s1, s2, s3):
    bufs = (b0, b1, b2, b3)
    gsems = (g0, g1, g2, g3)
    ssems = (s0, s1, s2, s3)
    wid = lax.axis_index("s") * NC + lax.axis_index("c")
    base = wid * PER_W
    pltpu.sync_copy(p_hbm, p_v)
    pltpu.sync_copy(
        x_hbm.at[pl.ds(wid * IDX_ROWS_PER_W, IDX_ROWS_PER_W)], idx_v
    )

    def issue_gathers(cc, b):
        for j in range(CHUNK_IDX):
            pltpu.async_copy(
                w_hbm.at[idx_v.at[cc * CHUNK_IDX + j]],
                bufs[b].at[pl.ds(j * IDX_MINOR, IDX_MINOR)],
                gsems[b],
            )

    def drain_gathers(b):
        for j in range(CHUNK_IDX):
            pltpu.make_async_copy(
                w_hbm.at[idx_v.at[j]],
                bufs[b].at[pl.ds(j * IDX_MINOR, IDX_MINOR)],
                gsems[b],
            ).wait()

    def wait_store(b):
        pltpu.make_async_copy(
            bufs[b], out_hbm.at[pl.ds(base, CHUNK_ROWS)], ssems[b]
        ).wait()

    issue_gathers(0, 0)

    @pl.loop(0, NCHUNK, step=NBUF)
    def _chunks(c):
        for b in range(NBUF):
            cc = c + b
            nb = (b + 1) % NBUF

            @pl.when(cc + 1 < NCHUNK)
            def _():
                @pl.when(cc >= NBUF - 1)
                def _():
                    wait_store(nb)
                issue_gathers(cc + 1, nb)

            drain_gathers(b)

            @pl.loop(0, SEQ)
            def _add(r):
                for d in range(DV):
                    sl = pl.ds(d * LANES, LANES)
                    bufs[b][r, sl] += p_v[r, sl]

            pltpu.async_copy(
                bufs[b],
                out_hbm.at[pl.ds(base + cc * CHUNK_ROWS, CHUNK_ROWS)],
                ssems[b],
            )

    for b in range(NBUF):
        wait_store(b)


_sc_kernel = functools.partial(
    pl.kernel,
    out_type=jax.ShapeDtypeStruct((NFLAT, D), jnp.float32),
    mesh=plsc.VectorSubcoreMesh(core_axis_name="c", subcore_axis_name="s"),
    scratch_types=[
        pltpu.VMEM((IDX_ROWS_PER_W, IDX_MINOR), jnp.int32),
        pltpu.VMEM((CHUNK_ROWS, D), jnp.float32),
        pltpu.VMEM((CHUNK_ROWS, D), jnp.float32),
        pltpu.VMEM((CHUNK_ROWS, D), jnp.float32),
        pltpu.VMEM((CHUNK_ROWS, D), jnp.float32),
        pltpu.VMEM((SEQ, D), jnp.float32),
        pltpu.SemaphoreType.DMA,
        pltpu.SemaphoreType.DMA,
        pltpu.SemaphoreType.DMA,
        pltpu.SemaphoreType.DMA,
        pltpu.SemaphoreType.DMA,
        pltpu.SemaphoreType.DMA,
        pltpu.SemaphoreType.DMA,
        pltpu.SemaphoreType.DMA,
    ],
    compiler_params=pltpu.CompilerParams(use_tc_tiling_on_sc=False),
)(_sc_body)


@jax.jit
def kernel(x, W, P):
    xf = x.reshape(NFLAT // IDX_MINOR, IDX_MINOR)
    out = _sc_kernel(xf, W, P)
    return out.reshape(BATCH, SEQ, D)


# speedup vs baseline: 4.1336x; 1.0022x over previous
import functools

import jax
import jax.numpy as jnp
from jax import lax
from jax.experimental import pallas as pl
from jax.experimental.pallas import tpu as pltpu
from jax.experimental.pallas import tpu_sc as plsc

BATCH = 4096
SEQ = 200
D = 64
NFLAT = BATCH * SEQ
NC, NS, LANES = 2, 16, 16
NW = NC * NS
PER_W = NFLAT // NW
IDX_MINOR = 100
IDX_ROWS_PER_W = PER_W // IDX_MINOR
CHUNK_ROWS = SEQ
CHUNK_IDX = CHUNK_ROWS // IDX_MINOR
NCHUNK = PER_W // CHUNK_ROWS
DV = D // LANES
NBUF = 4


def _sc_body(x_hbm, w_hbm, p_hbm, out_hbm,
             idx_v, b0, b1, b2, b3, p_v,
             g0, g1, g2, g3, s0, s1, s2, s3):
    bufs = (b0, b1, b2, b3)
    gsems = (g0, g1, g2, g3)
    ssems = (s0, s1, s2, s3)
    wid = lax.axis_index("s") * NC + lax.axis_index("c")
    base = wid * PER_W
    pltpu.sync_copy(p_hbm, p_v)
    pltpu.sync_copy(
        x_hbm.at[pl.ds(wid * IDX_ROWS_PER_W, IDX_ROWS_PER_W)], idx_v
    )

    def issue_gathers(cc, b):
        for j in range(CHUNK_IDX):
            pltpu.async_copy(
                w_hbm.at[idx_v.at[cc * CHUNK_IDX + j]],
                bufs[b].at[pl.ds(j * IDX_MINOR, IDX_MINOR)],
                gsems[b],
            )

    def drain_gathers(b):
        for j in range(CHUNK_IDX):
            pltpu.make_async_copy(
                w_hbm.at[idx_v.at[j]],
                bufs[b].at[pl.ds(j * IDX_MINOR, IDX_MINOR)],
                gsems[b],
            ).wait()

    def wait_store(b):
        pltpu.make_async_copy(bufs[b], out_hbm.at[0], ssems[b]).wait()

    issue_gathers(0, 0)

    @pl.loop(0, NCHUNK, step=NBUF)
    def _chunks(c):
        for b in range(NBUF):
            cc = c + b
            nb = (b + 1) % NBUF

            @pl.when(cc + 1 < NCHUNK)
            def _():
                @pl.when(cc >= NBUF - 1)
                def _():
                    wait_store(nb)
                issue_gathers(cc + 1, nb)

            drain_gathers(b)

            @pl.loop(0, SEQ)
            def _add(r):
                for d in range(DV):
                    sl = pl.ds(d * LANES, LANES)
                    bufs[b][r, sl] += p_v[r, sl]

            pltpu.async_copy(
                bufs[b],
                out_hbm.at[wid * NCHUNK + cc],
                ssems[b],
            )

    for b in range(NBUF):
        wait_store(b)


_sc_kernel = functools.partial(
    pl.kernel,
    out_type=jax.ShapeDtypeStruct((BATCH, SEQ, D), jnp.float32),
    mesh=plsc.VectorSubcoreMesh(core_axis_name="c", subcore_axis_name="s"),
    scratch_types=[
        pltpu.VMEM((IDX_ROWS_PER_W, IDX_MINOR), jnp.int32),
        pltpu.VMEM((CHUNK_ROWS, D), jnp.float32),
        pltpu.VMEM((CHUNK_ROWS, D), jnp.float32),
        pltpu.VMEM((CHUNK_ROWS, D), jnp.float32),
        pltpu.VMEM((CHUNK_ROWS, D), jnp.float32),
        pltpu.VMEM((SEQ, D), jnp.float32),
        pltpu.SemaphoreType.DMA,
        pltpu.SemaphoreType.DMA,
        pltpu.SemaphoreType.DMA,
        pltpu.SemaphoreType.DMA,
        pltpu.SemaphoreType.DMA,
        pltpu.SemaphoreType.DMA,
        pltpu.SemaphoreType.DMA,
        pltpu.SemaphoreType.DMA,
    ],
    compiler_params=pltpu.CompilerParams(use_tc_tiling_on_sc=False),
)(_sc_body)


@jax.jit
def kernel(x, W, P):
    xf = x.reshape(NFLAT // IDX_MINOR, IDX_MINOR)
    return _sc_kernel(xf, W, P)
